# baseline (speedup 1.0000x reference)
import functools

import jax
import jax.numpy as jnp
from jax import lax
from jax.experimental import pallas as pl
from jax.experimental.pallas import tpu as pltpu
from jax.experimental.pallas import tpu_sc as plsc

LIST_LEN = 16384
EMB = 128
N_CLS = 100000

NC = 2
NS = 16
NW = NC * NS
LANES = 16
ROWS_PER_W = LIST_LEN // NW
CHUNK = 128
CHUNKS_PER_TABLE = ROWS_PER_W // CHUNK
N_CHUNKS = 3 * CHUNKS_PER_TABLE
VREGS = EMB // LANES


UNROLL = 8


def _sc_gather_pool(item_list, entity_list, word_list,
                    item_tab, entity_tab, word_tab,
                    partials,
                    idx0, idx1, idx2, buf0, buf1, buf2, out_v, sem0, sem1, sem2, semi):
    wid = lax.axis_index("s") * NC + lax.axis_index("c")
    lists = (item_list, entity_list, word_list)
    tabs = (item_tab, entity_tab, word_tab)
    bufs = (buf0, buf1, buf2)
    sems = (sem0, sem1, sem2)

    base = wid * ROWS_PER_W
    idx_v = (idx0, idx1, idx2)
    stage = [pltpu.async_copy(lists[t].at[pl.ds(base, ROWS_PER_W)], idx_v[t], semi)
             for t in range(3)]
    for h in stage:
        h.wait()

    chunks = [(t, j) for t in range(3) for j in range(CHUNKS_PER_TABLE)]

    def fire(k):
        t, j = chunks[k]
        return pltpu.async_copy(
            tabs[t].at[idx_v[t].at[pl.ds(j * CHUNK, CHUNK)]], bufs[k % 3], sems[k % 3])

    handles = [None] * N_CHUNKS
    handles[0] = fire(0)
    handles[1] = fire(1)
    acc = tuple(jnp.zeros((LANES,), jnp.float32) for _ in range(VREGS))
    for k in range(N_CHUNKS):
        if k + 2 < N_CHUNKS:
            handles[k + 2] = fire(k + 2)
        handles[k].wait()
        buf = bufs[k % 3]

        def body(i, a, buf=buf):
            a = list(a)
            for dr in range(UNROLL):
                r = i * UNROLL + dr
                for c in range(VREGS):
                    a[c] = a[c] + buf[r, pl.ds(c * LANES, LANES)]
            return tuple(a)

        acc = lax.fori_loop(0, CHUNK // UNROLL, body, acc)

    for c in range(VREGS):
        out_v[pl.ds(c * LANES, LANES)] = acc[c]
    pltpu.sync_copy(out_v, partials.at[wid])


def _tc_matvec_body(p_ref, w_ref, o_ref):
    emb = jnp.sum(p_ref[...], axis=0, keepdims=True) * (1.0 / (3.0 * LIST_LEN))
    o_ref[...] = lax.dot_general(emb, w_ref[...], (((1,), (1,)), ((), ())),
                                 preferred_element_type=jnp.float32)


def kernel(item_list, entity_list, word_list, item_edger, entity_edger, word_edger,
           item_table, entity_table, word_table, W_cls, b_cls):
    item_idx = item_list.astype(jnp.int32)
    entity_idx = entity_list.astype(jnp.int32)
    word_idx = word_list.astype(jnp.int32)

    mesh = plsc.VectorSubcoreMesh(core_axis_name="c", subcore_axis_name="s")
    sc_call = functools.partial(
        pl.kernel,
        out_type=jax.ShapeDtypeStruct((NW, EMB), jnp.float32),
        mesh=mesh,
        scratch_types=[
            pltpu.VMEM((ROWS_PER_W,), jnp.int32),
            pltpu.VMEM((ROWS_PER_W,), jnp.int32),
            pltpu.VMEM((ROWS_PER_W,), jnp.int32),
            pltpu.VMEM((CHUNK, EMB), jnp.float32),
            pltpu.VMEM((CHUNK, EMB), jnp.float32),
            pltpu.VMEM((CHUNK, EMB), jnp.float32),
            pltpu.VMEM((EMB,), jnp.float32),
            pltpu.SemaphoreType.DMA,
            pltpu.SemaphoreType.DMA,
            pltpu.SemaphoreType.DMA,
            pltpu.SemaphoreType.DMA,
        ],
    )(_sc_gather_pool)
    partials = sc_call(item_idx, entity_idx, word_idx,
                       item_table, entity_table, word_table)

    bn = 12800
    n_blocks = pl.cdiv(N_CLS, bn)
    wt = W_cls.T
    out = pl.pallas_call(
        _tc_matvec_body,
        grid=(n_blocks,),
        in_specs=[
            pl.BlockSpec((NW, EMB), lambda i: (0, 0)),
            pl.BlockSpec((bn, EMB), lambda i: (i, 0)),
        ],
        out_specs=pl.BlockSpec((1, bn), lambda i: (0, i)),
        out_shape=jax.ShapeDtypeStruct((1, N_CLS), jnp.float32),
    )(partials, wt)
    return out + b_cls[None, :]


# speedup vs baseline: 1.0494x; 1.0494x over previous
import functools

import jax
import jax.numpy as jnp
from jax import lax
from jax.experimental import pallas as pl
from jax.experimental.pallas import tpu as pltpu
from jax.experimental.pallas import tpu_sc as plsc

LIST_LEN = 16384
EMB = 128
N_CLS = 100000

NC = 2
NS = 16
NW = NC * NS
LANES = 16
ROWS_PER_W = LIST_LEN // NW
CHUNK = 128
CHUNKS_PER_TABLE = ROWS_PER_W // CHUNK
N_CHUNKS = 3 * CHUNKS_PER_TABLE
VREGS = EMB // LANES


UNROLL = 8


def _sc_gather_pool(item_list, entity_list, word_list,
                    item_tab, entity_tab, word_tab,
                    partials,
                    idx0, idx1, idx2, buf0, buf1, buf2, out_v, sem0, sem1, sem2, semi):
    wid = lax.axis_index("s") * NC + lax.axis_index("c")
    lists = (item_list, entity_list, word_list)
    tabs = (item_tab, entity_tab, word_tab)
    bufs = (buf0, buf1, buf2)
    sems = (sem0, sem1, sem2)

    base = wid * ROWS_PER_W
    idx_v = (idx0, idx1, idx2)
    stage = [pltpu.async_copy(lists[t].at[pl.ds(base, ROWS_PER_W)], idx_v[t], semi)
             for t in range(3)]
    for h in stage:
        h.wait()

    chunks = [(t, j) for t in range(3) for j in range(CHUNKS_PER_TABLE)]

    def fire(k):
        t, j = chunks[k]
        return pltpu.async_copy(
            tabs[t].at[idx_v[t].at[pl.ds(j * CHUNK, CHUNK)]], bufs[k % 3], sems[k % 3])

    handles = [None] * N_CHUNKS
    handles[0] = fire(0)
    handles[1] = fire(1)
    acc = tuple(jnp.zeros((LANES,), jnp.float32) for _ in range(VREGS))
    for k in range(N_CHUNKS):
        if k + 2 < N_CHUNKS:
            handles[k + 2] = fire(k + 2)
        handles[k].wait()
        buf = bufs[k % 3]

        def body(i, a, buf=buf):
            a = list(a)
            for dr in range(UNROLL):
                r = i * UNROLL + dr
                for c in range(VREGS):
                    a[c] = a[c] + buf[r, pl.ds(c * LANES, LANES)]
            return tuple(a)

        acc = lax.fori_loop(0, CHUNK // UNROLL, body, acc)

    for c in range(VREGS):
        out_v[pl.ds(c * LANES, LANES)] = acc[c]
    pltpu.sync_copy(out_v, partials.at[wid])


def _tc_matvec_body(p_ref, w_ref, b_ref, o_ref):
    emb = jnp.sum(p_ref[...], axis=0, keepdims=True) * (1.0 / (3.0 * LIST_LEN))
    prod = lax.dot_general(emb, w_ref[...], (((1,), (1,)), ((), ())),
                           preferred_element_type=jnp.float32)
    o_ref[...] = prod + b_ref[...]


def kernel(item_list, entity_list, word_list, item_edger, entity_edger, word_edger,
           item_table, entity_table, word_table, W_cls, b_cls):
    item_idx = item_list.astype(jnp.int32)
    entity_idx = entity_list.astype(jnp.int32)
    word_idx = word_list.astype(jnp.int32)

    mesh = plsc.VectorSubcoreMesh(core_axis_name="c", subcore_axis_name="s")
    sc_call = functools.partial(
        pl.kernel,
        out_type=jax.ShapeDtypeStruct((NW, EMB), jnp.float32),
        mesh=mesh,
        scratch_types=[
            pltpu.VMEM((ROWS_PER_W,), jnp.int32),
            pltpu.VMEM((ROWS_PER_W,), jnp.int32),
            pltpu.VMEM((ROWS_PER_W,), jnp.int32),
            pltpu.VMEM((CHUNK, EMB), jnp.float32),
            pltpu.VMEM((CHUNK, EMB), jnp.float32),
            pltpu.VMEM((CHUNK, EMB), jnp.float32),
            pltpu.VMEM((EMB,), jnp.float32),
            pltpu.SemaphoreType.DMA,
            pltpu.SemaphoreType.DMA,
            pltpu.SemaphoreType.DMA,
            pltpu.SemaphoreType.DMA,
        ],
    )(_sc_gather_pool)
    partials = sc_call(item_idx, entity_idx, word_idx,
                       item_table, entity_table, word_table)

    bn = 12800
    n_blocks = pl.cdiv(N_CLS, bn)
    b2d = b_cls.reshape(1, N_CLS)
    wt = W_cls.T
    out = pl.pallas_call(
        _tc_matvec_body,
        grid=(n_blocks,),
        in_specs=[
            pl.BlockSpec((NW, EMB), lambda i: (0, 0)),
            pl.BlockSpec((bn, EMB), lambda i: (i, 0)),
            pl.BlockSpec((1, bn), lambda i: (0, i)),
        ],
        out_specs=pl.BlockSpec((1, bn), lambda i: (0, i)),
        out_shape=jax.ShapeDtypeStruct((1, N_CLS), jnp.float32),
    )(partials, wt, b2d)
    return out
